# Initial kernel scaffold; baseline (speedup 1.0000x reference)
import jax
import jax.numpy as jnp
from jax.experimental import pallas as pl


def kernel(x, edge_index, W1l, W1r, b1, gamma1, beta1, W2l, W2r, b2, gamma2, beta2):
    raise NotImplementedError("write your pallas kernel here")


# speedup vs baseline: 4.5336x; 4.5336x over previous
import functools

import jax
import jax.numpy as jnp
from jax import lax
from jax.experimental import pallas as pl
from jax.experimental.pallas import tpu as pltpu
from jax.experimental.pallas import tpu_sc as plsc

N = 10000
D = 128
EPS = 1e-5

NC = 2
NS = 16
NW = NC * NS
BATCH = 128

ACC_PER_TILE = 632
ACC_ROWS = NS * ACC_PER_TILE
DEG_PER_TILE = 640
DEG_ROWS = NS * DEG_PER_TILE
PAD_DST = 10008

_mesh = plsc.VectorSubcoreMesh(core_axis_name="c", subcore_axis_name="s")


def _sc_body(nb, h_hbm, src_hbm, dst_hbm, acc_out, deg_out,
             src_v, dst_v, rows_v, ones_v, acc_sh, deg_sh, sem):
    c = lax.axis_index("c")
    s = lax.axis_index("s")
    wid = s * NC + c

    zeros16 = jnp.zeros((16,), jnp.float32)
    ones16 = jnp.ones((16,), jnp.float32)

    def _zero_row_body(i, carry):
        for k in range(D // 16):
            rows_v[i, pl.ds(k * 16, 16)] = zeros16
        return carry

    lax.fori_loop(0, BATCH, _zero_row_body, 0)
    for k in range(BATCH // 16):
        ones_v[pl.ds(k * 16, 16)] = ones16

    base = s * ACC_PER_TILE
    n_full = ACC_PER_TILE // BATCH
    rem = ACC_PER_TILE - n_full * BATCH
    for k in range(n_full):
        pltpu.sync_copy(rows_v, acc_sh.at[pl.ds(base + k * BATCH, BATCH)])
    if rem:
        pltpu.sync_copy(rows_v.at[pl.ds(0, rem)],
                        acc_sh.at[pl.ds(base + n_full * BATCH, rem)])
    dbase = s * DEG_PER_TILE
    for k in range(DEG_PER_TILE // BATCH):
        pltpu.sync_copy(rows_v.at[0], deg_sh.at[pl.ds(dbase + k * BATCH, BATCH)])
    plsc.subcore_barrier()

    pltpu.sync_copy(src_hbm.at[wid], src_v)
    pltpu.sync_copy(dst_hbm.at[wid], dst_v)

    def _edge_body(j, carry):
        pltpu.async_copy(h_hbm.at[src_v.at[j]], rows_v, sem).wait()
        pltpu.sync_copy(rows_v, acc_sh.at[dst_v.at[j]], add=True)
        pltpu.sync_copy(ones_v, deg_sh.at[dst_v.at[j]], add=True)
        return carry

    lax.fori_loop(0, nb, _edge_body, 0)
    plsc.subcore_barrier()

    pltpu.sync_copy(acc_sh.at[pl.ds(base, ACC_PER_TILE)],
                    acc_out.at[c].at[pl.ds(base, ACC_PER_TILE)])
    pltpu.sync_copy(deg_sh.at[pl.ds(dbase, DEG_PER_TILE)],
                    deg_out.at[c].at[pl.ds(dbase, DEG_PER_TILE)])


def _make_sc_segment_sum(nb):
    return functools.partial(
        pl.kernel,
        mesh=_mesh,
        out_type=[
            jax.ShapeDtypeStruct((NC, ACC_ROWS, D), jnp.float32),
            jax.ShapeDtypeStruct((NC, DEG_ROWS), jnp.float32),
        ],
        scratch_types=[
            pltpu.VMEM((nb, BATCH), jnp.int32),
            pltpu.VMEM((nb, BATCH), jnp.int32),
            pltpu.VMEM((BATCH, D), jnp.float32),
            pltpu.VMEM((BATCH,), jnp.float32),
            pltpu.VMEM_SHARED((ACC_ROWS, D), jnp.float32),
            pltpu.VMEM_SHARED((DEG_ROWS,), jnp.float32),
            pltpu.SemaphoreType.DMA,
        ],
    )(functools.partial(_sc_body, nb))


def _dense_body(accA_ref, accB_ref, degA_ref, degB_ref, h_ref,
                Wl_ref, Wr_ref, b_ref, z_ref, stats_ref):
    i = pl.program_id(0)
    deg = degA_ref[...] + degB_ref[...]
    inv = 1.0 / jnp.maximum(deg, 1.0)
    S = (accA_ref[...] + accB_ref[...]) * inv
    z = lax.dot_general(S, Wl_ref[...], (((1,), (1,)), ((), ())),
                        preferred_element_type=jnp.float32)
    z = z + lax.dot_general(h_ref[...], Wr_ref[...], (((1,), (1,)), ((), ())),
                            preferred_element_type=jnp.float32)
    z = z + b_ref[...]
    z_ref[...] = z
    s0 = jnp.sum(z, axis=0, keepdims=True)
    s1 = jnp.sum(z * z, axis=0, keepdims=True)
    upd = jnp.concatenate([s0, s1, jnp.zeros((6, D), jnp.float32)], axis=0)

    @pl.when(i == 0)
    def _():
        stats_ref[...] = jnp.zeros((8, D), jnp.float32)

    stats_ref[...] += upd


def _bn_body(z_ref, stats_ref, gamma_ref, beta_ref, out_ref):
    st = stats_ref[...]
    mu = st[0:1, :] * (1.0 / N)
    var = st[1:2, :] * (1.0 / N) - mu * mu
    a = gamma_ref[...] * lax.rsqrt(var + EPS)
    out_ref[...] = jnp.maximum((z_ref[...] - mu) * a + beta_ref[...], 0.0)


def _dense_bn_relu(accA, accB, degA, degB, h, Wl, Wr, b, gamma, beta):
    R = 1000
    G = N // R
    row = lambda i: (i, 0)
    const = lambda i: (0, 0)
    z, stats = pl.pallas_call(
        _dense_body,
        grid=(G,),
        in_specs=[
            pl.BlockSpec((R, D), row),
            pl.BlockSpec((R, D), row),
            pl.BlockSpec((R, 1), row),
            pl.BlockSpec((R, 1), row),
            pl.BlockSpec((R, D), row),
            pl.BlockSpec((D, D), const),
            pl.BlockSpec((D, D), const),
            pl.BlockSpec((1, D), const),
        ],
        out_specs=[
            pl.BlockSpec((R, D), row),
            pl.BlockSpec((8, D), const),
        ],
        out_shape=[
            jax.ShapeDtypeStruct((N, D), jnp.float32),
            jax.ShapeDtypeStruct((8, D), jnp.float32),
        ],
    )(accA, accB, degA, degB, h, Wl, Wr, b)
    return pl.pallas_call(
        _bn_body,
        grid=(G,),
        in_specs=[
            pl.BlockSpec((R, D), row),
            pl.BlockSpec((8, D), const),
            pl.BlockSpec((1, D), const),
            pl.BlockSpec((1, D), const),
        ],
        out_specs=pl.BlockSpec((R, D), row),
        out_shape=jax.ShapeDtypeStruct((N, D), jnp.float32),
    )(z, stats, gamma, beta)


def kernel(x, edge_index, W1l, W1r, b1, gamma1, beta1, W2l, W2r, b2, gamma2, beta2):
    E = edge_index.shape[1]
    nb = -(-E // (NW * BATCH))
    epad = nb * NW * BATCH
    src = jnp.concatenate(
        [edge_index[0], jnp.zeros((epad - E,), jnp.int32)]).reshape(NW, nb, BATCH)
    dst = jnp.concatenate(
        [edge_index[1], jnp.full((epad - E,), PAD_DST, jnp.int32)]).reshape(NW, nb, BATCH)

    seg = _make_sc_segment_sum(nb)

    def layer(h, Wl, Wr, b, gamma, beta):
        acc, deg = seg(h, src, dst)
        return _dense_bn_relu(
            acc[0, :N], acc[1, :N],
            deg[0, :N, None], deg[1, :N, None],
            h, Wl, Wr, b.reshape(1, D), gamma.reshape(1, D), beta.reshape(1, D))

    h1 = layer(x, W1l, W1r, b1, gamma1, beta1)
    return layer(h1, W2l, W2r, b2, gamma2, beta2)
